# baseline (speedup 1.0000x reference)
import functools
import math

import jax
import jax.numpy as jnp
from jax import lax
from jax.experimental import pallas as pl
from jax.experimental.pallas import tpu as pltpu
from jax.experimental.pallas import tpu_sc as plsc

_HIDDEN = 768
_PATCH_NUM = 84
_SELECT_NUM = 42
_B = 4
_C = 12
_S = 1024
_H = 32
_PAD_SEL = 64

_HIGH = lax.Precision.HIGHEST


def _main_body(sn_ref, score_ref, row0_ref, w1_ref, w2_ref, hid_ref,
               out_ref, patch_out, gidx_out, row0s, sem_big, sem_row):
    big = pltpu.make_async_copy(hid_ref, out_ref, sem_big)
    big.start()

    score = score_ref[...]

    u = lax.bitcast_convert_type(score, jnp.uint32)
    neg = (u >> jnp.uint32(31)) > jnp.uint32(0)
    ukey = jnp.where(neg, ~u, u | jnp.uint32(0x80000000))

    def bs_body(i, m):
        cand = m | (jnp.uint32(0x80000000) >> i.astype(jnp.uint32))
        cnt = jnp.sum((ukey >= cand).astype(jnp.int32), axis=1, keepdims=True)
        return jnp.where(cnt >= _PATCH_NUM, cand, m)

    thr_key = lax.fori_loop(0, 32, bs_body, jnp.zeros((48, 1), jnp.uint32))

    gt = ukey > thr_key
    eq = ukey == thr_key
    cnt_gt = jnp.sum(gt.astype(jnp.int32), axis=1, keepdims=True)
    need = _PATCH_NUM - cnt_gt

    p_row = lax.broadcasted_iota(jnp.int32, (1024, 1024), 0)
    p_col = lax.broadcasted_iota(jnp.int32, (1024, 1024), 1)
    slt = jnp.where(p_row < p_col, 1.0, 0.0).astype(jnp.float32)

    eq_f = eq.astype(jnp.float32)
    eq_rank = lax.dot_general(eq_f, slt, (((1,), (0,)), ((), ())),
                              precision=_HIGH).astype(jnp.int32)
    mask = gt | (eq & (eq_rank < need))
    mask_f = mask.astype(jnp.float32)
    new_score = jnp.where(mask, score, score * 0.7)

    g_r = lax.broadcasted_iota(jnp.int32, (4, 48), 0)
    g_c = lax.broadcasted_iota(jnp.int32, (4, 48), 1)
    grp = jnp.where(g_c // _C == g_r, 1.0, 0.0).astype(jnp.float32)
    s1 = lax.dot_general(grp, new_score, (((1,), (0,)), ((), ())),
                         precision=_HIGH)
    count = lax.dot_general(grp, mask_f, (((1,), (0,)), ((), ())),
                            precision=_HIGH)
    pw = s1 * (1.0 / _C)

    thr = jnp.mean(s1, axis=1, keepdims=True)
    binary = (s1 > thr).astype(jnp.float32)
    m_arr = pw * binary

    lane = lax.broadcasted_iota(jnp.int32, (4, 1024), 1)
    mx = jnp.max(m_arr, axis=1, keepdims=True)
    idx_max = jnp.min(jnp.where(m_arr == mx, lane, 1024), axis=1,
                      keepdims=True)

    ai = (idx_max // _H).astype(jnp.float32)
    aj = (idx_max % _H).astype(jnp.float32)
    pi = (lane // _H).astype(jnp.float32)
    pj = (lane % _H).astype(jnp.float32)
    ri = (pi - ai) * (1.0 / _H)
    rj = (pj - aj) * (1.0 / _H)
    dist = jnp.sqrt(ri * ri + rj * rj)
    ang = (jnp.arctan2(rj, ri) * (1.0 / math.pi) + 1.0) * 0.5

    cw = jnp.sum(pw * dist, axis=1, keepdims=True)
    ca = jnp.sum(pw * ang, axis=1, keepdims=True)
    pw2 = pw * pw
    p_pos = jnp.sum(jnp.where(pw > 0, pw2, 0.0), axis=1, keepdims=True)
    p_neg = jnp.sum(jnp.where(pw < 0, pw2, 0.0), axis=1, keepdims=True)

    w1 = w1_ref[...]
    t = cw * w1[0:1, :] + ca * w1[1:2, :]
    v = t * jnp.where(t > 0, p_pos, p_neg)
    w = lax.dot_general(v, w2_ref[...], (((1,), (0,)), ((), ())),
                        precision=_HIGH)
    pw_anchor = jnp.sum(jnp.where(lane == idx_max, pw, 0.0), axis=1,
                        keepdims=True)
    z = pw_anchor * w
    sinfo = jnp.where(z >= 0, z, 0.2 * z)
    row0s[...] = row0_ref[...] + sinfo

    d_i = jnp.abs((p_row >> 5) - (p_col >> 5))
    d_j = jnp.abs((p_row & 31) - (p_col & 31))
    stencil = jnp.where((d_i <= 1) & (d_j <= 1), (2 - d_i) * (2 - d_j),
                        0).astype(jnp.float32)
    csm = lax.dot_general(count, stencil, (((1,), (0,)), ((), ())),
                          precision=_HIGH)
    ci = csm.astype(jnp.int32)
    key2 = ci * 1024 + (1023 - lane)

    keep = jnp.minimum(jnp.int32(_SELECT_NUM), sn_ref[0, 0])
    r_lane = lax.broadcasted_iota(jnp.int32, (1, _PAD_SEL), 1)
    pv = (lane[0:1, :] + 1).astype(jnp.float32)

    for b in range(_B):
        k2 = key2[b:b + 1, :]
        gtm = (k2 > k2.reshape(1024, 1)).astype(jnp.float32)
        rank = jnp.sum(gtm, axis=1, keepdims=True).astype(jnp.int32)
        onehot = (rank == r_lane).astype(jnp.float32)
        patch = lax.dot_general(pv, onehot, (((1,), (0,)), ((), ())),
                                precision=_HIGH)
        patch = jnp.where(r_lane < keep, patch.astype(jnp.int32), 0)
        patch_out[b:b + 1, :] = patch
        gidx_out[b:b + 1, :] = patch + b * (_S + 1)

    big.wait()
    for b in range(_B):
        pltpu.make_async_copy(row0s.at[b], out_ref.at[b, 0], sem_row).start()
    for b in range(_B):
        pltpu.make_async_copy(row0s.at[b], out_ref.at[b, 0], sem_row).wait()


def _gather_body(tab_ref, idx_ref, out_ref, idx_v, rows_v, sem):
    nc = 2
    wid = lax.axis_index("s") * nc + lax.axis_index("c")
    per = (_B * _PAD_SEL) // (nc * 16)
    base = wid * per
    pltpu.sync_copy(idx_ref.at[pl.ds(base, per)], idx_v)
    pltpu.async_copy(tab_ref.at[idx_v], rows_v, sem).wait()
    pltpu.sync_copy(rows_v, out_ref.at[pl.ds(base, per)])


def kernel(hidden_states, x, contribution, select_num, W1, W2):
    del contribution
    score = x[:, :, 0, 1:].reshape(_B * _C, _S)
    row0 = hidden_states[:, 0, :]
    sn = jnp.asarray(select_num, jnp.int32).reshape(1, 1)

    hidden_out, patch_pad, gidx = pl.pallas_call(
        _main_body,
        in_specs=[
            pl.BlockSpec(memory_space=pltpu.SMEM),
            pl.BlockSpec(memory_space=pltpu.VMEM),
            pl.BlockSpec(memory_space=pltpu.VMEM),
            pl.BlockSpec(memory_space=pltpu.VMEM),
            pl.BlockSpec(memory_space=pltpu.VMEM),
            pl.BlockSpec(memory_space=pl.ANY),
        ],
        out_specs=[
            pl.BlockSpec(memory_space=pl.ANY),
            pl.BlockSpec(memory_space=pltpu.VMEM),
            pl.BlockSpec(memory_space=pltpu.VMEM),
        ],
        out_shape=[
            jax.ShapeDtypeStruct((_B, _S + 1, _HIDDEN), jnp.float32),
            jax.ShapeDtypeStruct((_B, _PAD_SEL), jnp.int32),
            jax.ShapeDtypeStruct((_B, _PAD_SEL), jnp.int32),
        ],
        scratch_shapes=[
            pltpu.VMEM((_B, _HIDDEN), jnp.float32),
            pltpu.SemaphoreType.DMA,
            pltpu.SemaphoreType.DMA,
        ],
    )(sn, score, row0, W1, W2, hidden_states)

    mesh = plsc.VectorSubcoreMesh(core_axis_name="c", subcore_axis_name="s")
    gather = functools.partial(
        pl.kernel,
        mesh=mesh,
        out_type=jax.ShapeDtypeStruct((_B * _PAD_SEL, _HIDDEN), jnp.float32),
        scratch_types=[
            pltpu.VMEM(((_B * _PAD_SEL) // 32,), jnp.int32),
            pltpu.VMEM(((_B * _PAD_SEL) // 32, _HIDDEN), jnp.float32),
            pltpu.SemaphoreType.DMA,
        ],
    )(_gather_body)
    flat = gather(hidden_states.reshape(_B * (_S + 1), _HIDDEN),
                  gidx.reshape(_B * _PAD_SEL))
    sel = flat.reshape(_B, _PAD_SEL, _HIDDEN)[:, :_SELECT_NUM, :]

    patch_idx = patch_pad[:, :_SELECT_NUM]
    selected = jnp.where((patch_idx == 0)[:, :, None],
                         hidden_out[:, 0, :][:, None, :], sel)
    return hidden_out, selected, patch_idx


# speedup vs baseline: 2.4237x; 2.4237x over previous
import functools
import math

import jax
import jax.numpy as jnp
from jax import lax
from jax.experimental import pallas as pl
from jax.experimental.pallas import tpu as pltpu
from jax.experimental.pallas import tpu_sc as plsc

_HIDDEN = 768
_PATCH_NUM = 84
_SELECT_NUM = 42
_B = 4
_C = 12
_S = 1024
_H = 32
_PAD_SEL = 64
_CPB = 512
_NBLK = (_B * (_S + 1) + _CPB - 1) // _CPB

_HIGH = lax.Precision.HIGHEST


def _bf01(pred):
    return pred.astype(jnp.float32).astype(jnp.bfloat16)


def _compute_body(sn_ref, score_ref, row0_ref, w1_ref, w2_ref,
                  row0_out, patch_out, gidx_out):
    score = score_ref[...]
    ones_v = jnp.full((1024, 1), jnp.bfloat16(1))

    u = lax.bitcast_convert_type(score, jnp.uint32)
    neg = (u >> jnp.uint32(31)) > jnp.uint32(0)
    ukey = jnp.where(neg, ~u, u | jnp.uint32(0x80000000))

    def bs_body(i, m):
        cand = m | (jnp.uint32(0x80000000) >> i.astype(jnp.uint32))
        geb = _bf01(ukey >= cand)
        cnt = lax.dot_general(geb, ones_v, (((1,), (0,)), ((), ())),
                              preferred_element_type=jnp.float32)
        return jnp.where(cnt >= float(_PATCH_NUM), cand, m)

    thr_key = lax.fori_loop(0, 32, bs_body, jnp.zeros((48, 1), jnp.uint32))

    gt = ukey > thr_key
    eq = ukey == thr_key
    gtb = _bf01(gt)
    cnt_gt = lax.dot_general(gtb, ones_v, (((1,), (0,)), ((), ())),
                             preferred_element_type=jnp.float32)
    need = (float(_PATCH_NUM) - cnt_gt)

    p_row = lax.broadcasted_iota(jnp.int32, (1024, 1024), 0)
    p_col = lax.broadcasted_iota(jnp.int32, (1024, 1024), 1)
    slt = _bf01(p_row < p_col)

    eq_b = _bf01(eq)
    eq_rank = lax.dot_general(eq_b, slt, (((1,), (0,)), ((), ())),
                              preferred_element_type=jnp.float32)
    mask = gt | (eq & (eq_rank < need))
    mask_b = _bf01(mask)
    new_score = jnp.where(mask, score, score * 0.7)

    g_r = lax.broadcasted_iota(jnp.int32, (4, 48), 0)
    g_c = lax.broadcasted_iota(jnp.int32, (4, 48), 1)
    grp = jnp.where(g_c // _C == g_r, 1.0, 0.0).astype(jnp.float32)
    s1 = lax.dot_general(grp, new_score, (((1,), (0,)), ((), ())),
                         precision=_HIGH)
    grp_b = grp.astype(jnp.bfloat16)
    count = lax.dot_general(grp_b, mask_b, (((1,), (0,)), ((), ())),
                            preferred_element_type=jnp.float32)
    pw = s1 * (1.0 / _C)

    thr = jnp.mean(s1, axis=1, keepdims=True)
    binary = (s1 > thr).astype(jnp.float32)
    m_arr = pw * binary

    lane = lax.broadcasted_iota(jnp.int32, (4, 1024), 1)
    mx = jnp.max(m_arr, axis=1, keepdims=True)
    idx_max = jnp.min(jnp.where(m_arr == mx, lane, 1024), axis=1,
                      keepdims=True)

    ai = (idx_max // _H).astype(jnp.float32)
    aj = (idx_max % _H).astype(jnp.float32)
    pi = (lane // _H).astype(jnp.float32)
    pj = (lane % _H).astype(jnp.float32)
    ri = (pi - ai) * (1.0 / _H)
    rj = (pj - aj) * (1.0 / _H)
    dist = jnp.sqrt(ri * ri + rj * rj)
    ang = (jnp.arctan2(rj, ri) * (1.0 / math.pi) + 1.0) * 0.5

    cw = jnp.sum(pw * dist, axis=1, keepdims=True)
    ca = jnp.sum(pw * ang, axis=1, keepdims=True)
    pw2 = pw * pw
    p_pos = jnp.sum(jnp.where(pw > 0, pw2, 0.0), axis=1, keepdims=True)
    p_neg = jnp.sum(jnp.where(pw < 0, pw2, 0.0), axis=1, keepdims=True)

    w1 = w1_ref[...]
    t = cw * w1[0:1, :] + ca * w1[1:2, :]
    v = t * jnp.where(t > 0, p_pos, p_neg)
    w = lax.dot_general(v, w2_ref[...], (((1,), (0,)), ((), ())),
                        precision=_HIGH)
    pw_anchor = jnp.sum(jnp.where(lane == idx_max, pw, 0.0), axis=1,
                        keepdims=True)
    z = pw_anchor * w
    sinfo = jnp.where(z >= 0, z, 0.2 * z)
    row0_out[...] = row0_ref[...] + sinfo

    d_i = jnp.abs((p_row >> 5) - (p_col >> 5))
    d_j = jnp.abs((p_row & 31) - (p_col & 31))
    sten_b = jnp.where((d_i <= 1) & (d_j <= 1), (2 - d_i) * (2 - d_j),
                       0).astype(jnp.bfloat16)
    count_b = count.astype(jnp.bfloat16)
    csm = lax.dot_general(count_b, sten_b, (((1,), (0,)), ((), ())),
                          preferred_element_type=jnp.float32)
    ci = csm.astype(jnp.int32)
    key2 = ci * 1024 + (1023 - lane)

    keep = jnp.minimum(jnp.int32(_SELECT_NUM), sn_ref[0, 0])
    r_lane = lax.broadcasted_iota(jnp.int32, (1, _PAD_SEL), 1)
    pv = (lane[0:1, :] + 1).astype(jnp.float32)
    ones_bt = jnp.full((1, 1024), jnp.bfloat16(1))

    for b in range(_B):
        k2 = key2[b:b + 1, :]
        gtm = _bf01(k2 > k2.reshape(1024, 1))
        rank = lax.dot_general(gtm, ones_v, (((1,), (0,)), ((), ())),
                               preferred_element_type=jnp.float32)
        rank = rank.astype(jnp.int32)
        onehot = (rank == r_lane).astype(jnp.float32)
        patch = lax.dot_general(pv, onehot, (((1,), (0,)), ((), ())),
                                precision=_HIGH)
        patch = jnp.where(r_lane < keep, patch.astype(jnp.int32), 0)
        patch_out[b:b + 1, :] = patch
        gidx_out[b:b + 1, :] = patch + b * (_S + 1)


def _copy_body(row0_ref, hid_ref, out_ref):
    out_ref[...] = hid_ref[...]
    blk = pl.program_id(0)
    bb = blk // 2

    @pl.when((blk % 2 == 0) & (blk < 2 * _B))
    def _():
        out_ref[pl.ds(bb, 1), :] = row0_ref[pl.ds(bb, 1), :]


def _gather_body(tab_ref, idx_ref, out_ref, idx_v, rows_v, sem):
    nc = 2
    wid = lax.axis_index("s") * nc + lax.axis_index("c")
    per = (_B * _PAD_SEL) // (nc * 16)
    base = wid * per
    pltpu.sync_copy(idx_ref.at[pl.ds(base, per)], idx_v)
    pltpu.async_copy(tab_ref.at[idx_v], rows_v, sem).wait()
    pltpu.sync_copy(rows_v, out_ref.at[pl.ds(base, per)])


def kernel(hidden_states, x, contribution, select_num, W1, W2):
    del contribution
    score = x[:, :, 0, 1:].reshape(_B * _C, _S)
    row0 = hidden_states[:, 0, :]
    sn = jnp.asarray(select_num, jnp.int32).reshape(1, 1)
    hid_flat = hidden_states.reshape(_B * (_S + 1), _HIDDEN)

    row0_new, patch_pad, gidx = pl.pallas_call(
        _compute_body,
        in_specs=[pl.BlockSpec(memory_space=pltpu.SMEM)] +
                 [pl.BlockSpec(memory_space=pltpu.VMEM)] * 4,
        out_specs=[pl.BlockSpec(memory_space=pltpu.VMEM)] * 3,
        out_shape=[
            jax.ShapeDtypeStruct((_B, _HIDDEN), jnp.float32),
            jax.ShapeDtypeStruct((_B, _PAD_SEL), jnp.int32),
            jax.ShapeDtypeStruct((_B, _PAD_SEL), jnp.int32),
        ],
    )(sn, score, row0, W1, W2)

    hidden_out_flat = pl.pallas_call(
        _copy_body,
        grid=(_NBLK,),
        in_specs=[
            pl.BlockSpec((_B, _HIDDEN), lambda i: (0, 0)),
            pl.BlockSpec((_CPB, _HIDDEN), lambda i: (i, 0)),
        ],
        out_specs=pl.BlockSpec((_CPB, _HIDDEN), lambda i: (i, 0)),
        out_shape=jax.ShapeDtypeStruct((_B * (_S + 1), _HIDDEN), jnp.float32),
    )(row0_new, hid_flat)
    hidden_out = hidden_out_flat.reshape(_B, _S + 1, _HIDDEN)

    mesh = plsc.VectorSubcoreMesh(core_axis_name="c", subcore_axis_name="s")
    gather = functools.partial(
        pl.kernel,
        mesh=mesh,
        out_type=jax.ShapeDtypeStruct((_B * _PAD_SEL, _HIDDEN), jnp.float32),
        scratch_types=[
            pltpu.VMEM(((_B * _PAD_SEL) // 32,), jnp.int32),
            pltpu.VMEM(((_B * _PAD_SEL) // 32, _HIDDEN), jnp.float32),
            pltpu.SemaphoreType.DMA,
        ],
    )(_gather_body)
    flat = gather(hid_flat, gidx.reshape(_B * _PAD_SEL))
    sel = flat.reshape(_B, _PAD_SEL, _HIDDEN)[:, :_SELECT_NUM, :]

    patch_idx = patch_pad[:, :_SELECT_NUM]
    selected = jnp.where((patch_idx == 0)[:, :, None],
                         hidden_out[:, 0, :][:, None, :], sel)
    return hidden_out, selected, patch_idx


# speedup vs baseline: 4.0652x; 1.6772x over previous
import functools
import math

import jax
import jax.numpy as jnp
from jax import lax
from jax.experimental import pallas as pl
from jax.experimental.pallas import tpu as pltpu
from jax.experimental.pallas import tpu_sc as plsc

_HIDDEN = 768
_PATCH_NUM = 84
_SELECT_NUM = 42
_B = 4
_C = 12
_S = 1024
_H = 32
_PAD_SEL = 64
_RPB = 128
_NROWBLK = (_S + 1 + _RPB - 1) // _RPB

_HIGH = lax.Precision.HIGHEST


def _bf01(pred):
    return pred.astype(jnp.float32).astype(jnp.bfloat16)


def _compute_body(sn_ref, score_ref, row0_ref, w1_ref, w2_ref,
                  row0_out, patch_out, gidx_out):
    score = score_ref[...]
    ones_v = jnp.full((1024, 1), jnp.bfloat16(1))

    u = lax.bitcast_convert_type(score, jnp.uint32)
    neg = (u >> jnp.uint32(31)) > jnp.uint32(0)
    ukey = jnp.where(neg, ~u, u | jnp.uint32(0x80000000))

    def bs_body(i, m):
        cand = m | (jnp.uint32(0x80000000) >> i.astype(jnp.uint32))
        geb = _bf01(ukey >= cand)
        cnt = lax.dot_general(geb, ones_v, (((1,), (0,)), ((), ())),
                              preferred_element_type=jnp.float32)
        return jnp.where(cnt >= float(_PATCH_NUM), cand, m)

    thr_key = lax.fori_loop(0, 32, bs_body, jnp.zeros((48, 1), jnp.uint32))

    gt = ukey > thr_key
    eq = ukey == thr_key
    gtb = _bf01(gt)
    cnt_gt = lax.dot_general(gtb, ones_v, (((1,), (0,)), ((), ())),
                             preferred_element_type=jnp.float32)
    need = (float(_PATCH_NUM) - cnt_gt)

    p_row = lax.broadcasted_iota(jnp.int32, (1024, 1024), 0)
    p_col = lax.broadcasted_iota(jnp.int32, (1024, 1024), 1)
    slt = _bf01(p_row < p_col)

    eq_b = _bf01(eq)
    eq_rank = lax.dot_general(eq_b, slt, (((1,), (0,)), ((), ())),
                              preferred_element_type=jnp.float32)
    mask = gt | (eq & (eq_rank < need))
    mask_b = _bf01(mask)
    new_score = jnp.where(mask, score, score * 0.7)

    g_r = lax.broadcasted_iota(jnp.int32, (4, 48), 0)
    g_c = lax.broadcasted_iota(jnp.int32, (4, 48), 1)
    grp = jnp.where(g_c // _C == g_r, 1.0, 0.0).astype(jnp.float32)
    s1 = lax.dot_general(grp, new_score, (((1,), (0,)), ((), ())),
                         precision=_HIGH)
    grp_b = grp.astype(jnp.bfloat16)
    count = lax.dot_general(grp_b, mask_b, (((1,), (0,)), ((), ())),
                            preferred_element_type=jnp.float32)
    pw = s1 * (1.0 / _C)

    thr = jnp.mean(s1, axis=1, keepdims=True)
    binary = (s1 > thr).astype(jnp.float32)
    m_arr = pw * binary

    lane = lax.broadcasted_iota(jnp.int32, (4, 1024), 1)
    mx = jnp.max(m_arr, axis=1, keepdims=True)
    idx_max = jnp.min(jnp.where(m_arr == mx, lane, 1024), axis=1,
                      keepdims=True)

    ai = (idx_max // _H).astype(jnp.float32)
    aj = (idx_max % _H).astype(jnp.float32)
    pi = (lane // _H).astype(jnp.float32)
    pj = (lane % _H).astype(jnp.float32)
    ri = (pi - ai) * (1.0 / _H)
    rj = (pj - aj) * (1.0 / _H)
    dist = jnp.sqrt(ri * ri + rj * rj)
    ang = (jnp.arctan2(rj, ri) * (1.0 / math.pi) + 1.0) * 0.5

    cw = jnp.sum(pw * dist, axis=1, keepdims=True)
    ca = jnp.sum(pw * ang, axis=1, keepdims=True)
    pw2 = pw * pw
    p_pos = jnp.sum(jnp.where(pw > 0, pw2, 0.0), axis=1, keepdims=True)
    p_neg = jnp.sum(jnp.where(pw < 0, pw2, 0.0), axis=1, keepdims=True)

    w1 = w1_ref[...]
    t = cw * w1[0:1, :] + ca * w1[1:2, :]
    v = t * jnp.where(t > 0, p_pos, p_neg)
    w = lax.dot_general(v, w2_ref[...], (((1,), (0,)), ((), ())),
                        precision=_HIGH)
    pw_anchor = jnp.sum(jnp.where(lane == idx_max, pw, 0.0), axis=1,
                        keepdims=True)
    z = pw_anchor * w
    sinfo = jnp.where(z >= 0, z, 0.2 * z)
    row0_out[...] = row0_ref[...] + sinfo

    d_i = jnp.abs((p_row >> 5) - (p_col >> 5))
    d_j = jnp.abs((p_row & 31) - (p_col & 31))
    sten_b = jnp.where((d_i <= 1) & (d_j <= 1), (2 - d_i) * (2 - d_j),
                       0).astype(jnp.bfloat16)
    count_b = count.astype(jnp.bfloat16)
    csm = lax.dot_general(count_b, sten_b, (((1,), (0,)), ((), ())),
                          preferred_element_type=jnp.float32)
    ci = csm.astype(jnp.int32)
    key2 = ci * 1024 + (1023 - lane)

    keep = jnp.minimum(jnp.int32(_SELECT_NUM), sn_ref[0, 0])
    r_lane = lax.broadcasted_iota(jnp.int32, (1, _PAD_SEL), 1)
    pv = (lane[0:1, :] + 1).astype(jnp.float32)
    ones_bt = jnp.full((1, 1024), jnp.bfloat16(1))

    for b in range(_B):
        k2 = key2[b:b + 1, :]
        gtm = _bf01(k2 > k2.reshape(1024, 1))
        rank = lax.dot_general(gtm, ones_v, (((1,), (0,)), ((), ())),
                               preferred_element_type=jnp.float32)
        rank = rank.astype(jnp.int32)
        onehot = (rank == r_lane).astype(jnp.float32)
        patch = lax.dot_general(pv, onehot, (((1,), (0,)), ((), ())),
                                precision=_HIGH)
        patch = jnp.where(r_lane < keep, patch.astype(jnp.int32), 0)
        patch_out[b:b + 1, :] = patch
        gidx_out[b:b + 1, :] = patch + b * (_S + 1)


def _copy_body(row0_ref, hid_ref, out_ref):
    out_ref[...] = hid_ref[...]
    b = pl.program_id(0)
    j = pl.program_id(1)

    @pl.when(j == 0)
    def _():
        out_ref[0, 0:1, :] = row0_ref[pl.ds(b, 1), :]


def _gather_body(tab_ref, idx_ref, out_ref, idx_v, rows_v, sem):
    nc = 2
    wid = lax.axis_index("s") * nc + lax.axis_index("c")
    per = (_B * _PAD_SEL) // (nc * 16)
    base = wid * per
    pltpu.sync_copy(idx_ref.at[pl.ds(base, per)], idx_v)
    pltpu.async_copy(tab_ref.at[idx_v], rows_v, sem).wait()
    pltpu.sync_copy(rows_v, out_ref.at[pl.ds(base, per)])


def kernel(hidden_states, x, contribution, select_num, W1, W2):
    del contribution
    score = x[:, :, 0, 1:].reshape(_B * _C, _S)
    row0 = hidden_states[:, 0, :]
    sn = jnp.asarray(select_num, jnp.int32).reshape(1, 1)
    hid_flat = hidden_states.reshape(_B * (_S + 1), _HIDDEN)

    row0_new, patch_pad, gidx = pl.pallas_call(
        _compute_body,
        in_specs=[pl.BlockSpec(memory_space=pltpu.SMEM)] +
                 [pl.BlockSpec(memory_space=pltpu.VMEM)] * 4,
        out_specs=[pl.BlockSpec(memory_space=pltpu.VMEM)] * 3,
        out_shape=[
            jax.ShapeDtypeStruct((_B, _HIDDEN), jnp.float32),
            jax.ShapeDtypeStruct((_B, _PAD_SEL), jnp.int32),
            jax.ShapeDtypeStruct((_B, _PAD_SEL), jnp.int32),
        ],
    )(sn, score, row0, W1, W2)

    hidden_out = pl.pallas_call(
        _copy_body,
        grid=(_B, _NROWBLK),
        in_specs=[
            pl.BlockSpec((_B, _HIDDEN), lambda b, j: (0, 0)),
            pl.BlockSpec((1, _RPB, _HIDDEN), lambda b, j: (b, j, 0)),
        ],
        out_specs=pl.BlockSpec((1, _RPB, _HIDDEN), lambda b, j: (b, j, 0)),
        out_shape=jax.ShapeDtypeStruct((_B, _S + 1, _HIDDEN), jnp.float32),
    )(row0_new, hidden_states)

    mesh = plsc.VectorSubcoreMesh(core_axis_name="c", subcore_axis_name="s")
    gather = functools.partial(
        pl.kernel,
        mesh=mesh,
        out_type=jax.ShapeDtypeStruct((_B * _PAD_SEL, _HIDDEN), jnp.float32),
        scratch_types=[
            pltpu.VMEM(((_B * _PAD_SEL) // 32,), jnp.int32),
            pltpu.VMEM(((_B * _PAD_SEL) // 32, _HIDDEN), jnp.float32),
            pltpu.SemaphoreType.DMA,
        ],
    )(_gather_body)
    flat = gather(hid_flat, gidx.reshape(_B * _PAD_SEL))
    sel = flat.reshape(_B, _PAD_SEL, _HIDDEN)[:, :_SELECT_NUM, :]

    patch_idx = patch_pad[:, :_SELECT_NUM]
    selected = jnp.where((patch_idx == 0)[:, :, None],
                         hidden_out[:, 0, :][:, None, :], sel)
    return hidden_out, selected, patch_idx


# speedup vs baseline: 6.1142x; 1.5040x over previous
import functools
import math

import jax
import jax.numpy as jnp
from jax import lax
from jax.experimental import pallas as pl
from jax.experimental.pallas import tpu as pltpu
from jax.experimental.pallas import tpu_sc as plsc

_HIDDEN = 768
_PATCH_NUM = 84
_SELECT_NUM = 42
_B = 4
_C = 12
_S = 1024
_H = 32
_PAD_SEL = 64

_HIGH = lax.Precision.HIGHEST


def _bf01(pred):
    return pred.astype(jnp.float32).astype(jnp.bfloat16)


def _compute_body(sn_ref, score_ref, row0_ref, w1_ref, w2_ref,
                  row0_out, patch_out, gidx_out):
    score = score_ref[...]
    ones_v = jnp.full((1024, 1), jnp.bfloat16(1))

    u = lax.bitcast_convert_type(score, jnp.uint32)
    neg = (u >> jnp.uint32(31)) > jnp.uint32(0)
    ukey = jnp.where(neg, ~u, u | jnp.uint32(0x80000000))

    def bs_body(i, m):
        cand = m | (jnp.uint32(0x80000000) >> i.astype(jnp.uint32))
        geb = _bf01(ukey >= cand)
        cnt = lax.dot_general(geb, ones_v, (((1,), (0,)), ((), ())),
                              preferred_element_type=jnp.float32)
        return jnp.where(cnt >= float(_PATCH_NUM), cand, m)

    thr_key = lax.fori_loop(0, 32, bs_body, jnp.zeros((48, 1), jnp.uint32))

    gt = ukey > thr_key
    eq = ukey == thr_key
    gtb = _bf01(gt)
    cnt_gt = lax.dot_general(gtb, ones_v, (((1,), (0,)), ((), ())),
                             preferred_element_type=jnp.float32)
    need = (float(_PATCH_NUM) - cnt_gt)

    p_row = lax.broadcasted_iota(jnp.int32, (1024, 1024), 0)
    p_col = lax.broadcasted_iota(jnp.int32, (1024, 1024), 1)
    slt = _bf01(p_row < p_col)

    eq_b = _bf01(eq)
    eq_rank = lax.dot_general(eq_b, slt, (((1,), (0,)), ((), ())),
                              preferred_element_type=jnp.float32)
    mask = gt | (eq & (eq_rank < need))
    mask_b = _bf01(mask)
    new_score = jnp.where(mask, score, score * 0.7)

    g_r = lax.broadcasted_iota(jnp.int32, (4, 48), 0)
    g_c = lax.broadcasted_iota(jnp.int32, (4, 48), 1)
    grp = jnp.where(g_c // _C == g_r, 1.0, 0.0).astype(jnp.float32)
    s1 = lax.dot_general(grp, new_score, (((1,), (0,)), ((), ())),
                         precision=_HIGH)
    grp_b = grp.astype(jnp.bfloat16)
    count = lax.dot_general(grp_b, mask_b, (((1,), (0,)), ((), ())),
                            preferred_element_type=jnp.float32)
    pw = s1 * (1.0 / _C)

    thr = jnp.mean(s1, axis=1, keepdims=True)
    binary = (s1 > thr).astype(jnp.float32)
    m_arr = pw * binary

    lane = lax.broadcasted_iota(jnp.int32, (4, 1024), 1)
    mx = jnp.max(m_arr, axis=1, keepdims=True)
    idx_max = jnp.min(jnp.where(m_arr == mx, lane, 1024), axis=1,
                      keepdims=True)

    ai = (idx_max // _H).astype(jnp.float32)
    aj = (idx_max % _H).astype(jnp.float32)
    pi = (lane // _H).astype(jnp.float32)
    pj = (lane % _H).astype(jnp.float32)
    ri = (pi - ai) * (1.0 / _H)
    rj = (pj - aj) * (1.0 / _H)
    dist = jnp.sqrt(ri * ri + rj * rj)
    ang = (jnp.arctan2(rj, ri) * (1.0 / math.pi) + 1.0) * 0.5

    cw = jnp.sum(pw * dist, axis=1, keepdims=True)
    ca = jnp.sum(pw * ang, axis=1, keepdims=True)
    pw2 = pw * pw
    p_pos = jnp.sum(jnp.where(pw > 0, pw2, 0.0), axis=1, keepdims=True)
    p_neg = jnp.sum(jnp.where(pw < 0, pw2, 0.0), axis=1, keepdims=True)

    w1 = w1_ref[...]
    t = cw * w1[0:1, :] + ca * w1[1:2, :]
    v = t * jnp.where(t > 0, p_pos, p_neg)
    w = lax.dot_general(v, w2_ref[...], (((1,), (0,)), ((), ())),
                        precision=_HIGH)
    pw_anchor = jnp.sum(jnp.where(lane == idx_max, pw, 0.0), axis=1,
                        keepdims=True)
    z = pw_anchor * w
    sinfo = jnp.where(z >= 0, z, 0.2 * z)
    row0_out[...] = row0_ref[...] + sinfo

    d_i = jnp.abs((p_row >> 5) - (p_col >> 5))
    d_j = jnp.abs((p_row & 31) - (p_col & 31))
    sten_b = jnp.where((d_i <= 1) & (d_j <= 1), (2 - d_i) * (2 - d_j),
                       0).astype(jnp.bfloat16)
    count_b = count.astype(jnp.bfloat16)
    csm = lax.dot_general(count_b, sten_b, (((1,), (0,)), ((), ())),
                          preferred_element_type=jnp.float32)
    ci = csm.astype(jnp.int32)
    key2 = ci * 1024 + (1023 - lane)

    keep = jnp.minimum(jnp.int32(_SELECT_NUM), sn_ref[0, 0])
    r_lane = lax.broadcasted_iota(jnp.int32, (1, _PAD_SEL), 1)
    pv = (lane[0:1, :] + 1).astype(jnp.float32)
    ones_bt = jnp.full((1, 1024), jnp.bfloat16(1))

    for b in range(_B):
        k2 = key2[b:b + 1, :]
        gtm = _bf01(k2 > k2.reshape(1024, 1))
        rank = lax.dot_general(gtm, ones_v, (((1,), (0,)), ((), ())),
                               preferred_element_type=jnp.float32)
        rank = rank.astype(jnp.int32)
        onehot = (rank == r_lane).astype(jnp.float32)
        patch = lax.dot_general(pv, onehot, (((1,), (0,)), ((), ())),
                                precision=_HIGH)
        patch = jnp.where(r_lane < keep, patch.astype(jnp.int32), 0)
        patch_out[b:b + 1, :] = patch
        gidx_out[b:b + 1, :] = patch + b * (_S + 1)


def _gather_body(tab_ref, idx_ref, out_ref, idx_v, rows_v, sem):
    nc = 2
    wid = lax.axis_index("s") * nc + lax.axis_index("c")
    per = (_B * _PAD_SEL) // (nc * 16)
    base = wid * per
    pltpu.sync_copy(idx_ref.at[pl.ds(base, per)], idx_v)
    pltpu.async_copy(tab_ref.at[idx_v], rows_v, sem).wait()
    pltpu.sync_copy(rows_v, out_ref.at[pl.ds(base, per)])


def kernel(hidden_states, x, contribution, select_num, W1, W2):
    del contribution
    score = x[:, :, 0, 1:].reshape(_B * _C, _S)
    row0 = hidden_states[:, 0, :]
    sn = jnp.asarray(select_num, jnp.int32).reshape(1, 1)
    hid_flat = hidden_states.reshape(_B * (_S + 1), _HIDDEN)

    row0_new, patch_pad, gidx = pl.pallas_call(
        _compute_body,
        in_specs=[pl.BlockSpec(memory_space=pltpu.SMEM)] +
                 [pl.BlockSpec(memory_space=pltpu.VMEM)] * 4,
        out_specs=[pl.BlockSpec(memory_space=pltpu.VMEM)] * 3,
        out_shape=[
            jax.ShapeDtypeStruct((_B, _HIDDEN), jnp.float32),
            jax.ShapeDtypeStruct((_B, _PAD_SEL), jnp.int32),
            jax.ShapeDtypeStruct((_B, _PAD_SEL), jnp.int32),
        ],
    )(sn, score, row0, W1, W2)

    hidden_out = hidden_states.at[:, 0, :].set(row0_new)

    mesh = plsc.VectorSubcoreMesh(core_axis_name="c", subcore_axis_name="s")
    gather = functools.partial(
        pl.kernel,
        mesh=mesh,
        out_type=jax.ShapeDtypeStruct((_B * _PAD_SEL, _HIDDEN), jnp.float32),
        scratch_types=[
            pltpu.VMEM(((_B * _PAD_SEL) // 32,), jnp.int32),
            pltpu.VMEM(((_B * _PAD_SEL) // 32, _HIDDEN), jnp.float32),
            pltpu.SemaphoreType.DMA,
        ],
    )(_gather_body)
    flat = gather(hid_flat, gidx.reshape(_B * _PAD_SEL))
    sel = flat.reshape(_B, _PAD_SEL, _HIDDEN)[:, :_SELECT_NUM, :]

    patch_idx = patch_pad[:, :_SELECT_NUM]
    selected = jnp.where((patch_idx == 0)[:, :, None],
                         hidden_out[:, 0, :][:, None, :], sel)
    return hidden_out, selected, patch_idx


# speedup vs baseline: 6.4556x; 1.0558x over previous
import functools
import math

import jax
import jax.numpy as jnp
from jax import lax
from jax.experimental import pallas as pl
from jax.experimental.pallas import tpu as pltpu
from jax.experimental.pallas import tpu_sc as plsc

_HIDDEN = 768
_PATCH_NUM = 84
_SELECT_NUM = 42
_B = 4
_C = 12
_S = 1024
_H = 32
_PAD_SEL = 64

_HIGH = lax.Precision.HIGHEST


def _bf01(pred):
    return pred.astype(jnp.float32).astype(jnp.bfloat16)


def _compute_body(sn_ref, score_ref, row0_ref, w1_ref, w2_ref,
                  row0_out, patch_out, gidx_out):
    score = score_ref[...]
    ones_v = jnp.full((1024, 1), jnp.bfloat16(1))

    u = lax.bitcast_convert_type(score, jnp.uint32)
    neg = (u >> jnp.uint32(31)) > jnp.uint32(0)
    ukey = jnp.where(neg, ~u, u | jnp.uint32(0x80000000))

    def bs_body(i, m):
        cand = m | (jnp.uint32(0x80000000) >> i.astype(jnp.uint32))
        cnt = jnp.sum((ukey >= cand).astype(jnp.int32), axis=1, keepdims=True)
        return jnp.where(cnt >= _PATCH_NUM, cand, m)

    thr_key = lax.fori_loop(0, 32, bs_body, jnp.zeros((48, 1), jnp.uint32))

    gt = ukey > thr_key
    eq = ukey == thr_key
    gtb = _bf01(gt)
    cnt_gt = lax.dot_general(gtb, ones_v, (((1,), (0,)), ((), ())),
                             preferred_element_type=jnp.float32)
    need = (float(_PATCH_NUM) - cnt_gt)

    p_row = lax.broadcasted_iota(jnp.int32, (1024, 1024), 0)
    p_col = lax.broadcasted_iota(jnp.int32, (1024, 1024), 1)
    slt = _bf01(p_row < p_col)

    eq_b = _bf01(eq)
    eq_rank = lax.dot_general(eq_b, slt, (((1,), (0,)), ((), ())),
                              preferred_element_type=jnp.float32)
    mask = gt | (eq & (eq_rank < need))
    mask_b = _bf01(mask)
    new_score = jnp.where(mask, score, score * 0.7)

    g_r = lax.broadcasted_iota(jnp.int32, (4, 48), 0)
    g_c = lax.broadcasted_iota(jnp.int32, (4, 48), 1)
    grp = jnp.where(g_c // _C == g_r, 1.0, 0.0).astype(jnp.float32)
    s1 = lax.dot_general(grp, new_score, (((1,), (0,)), ((), ())),
                         precision=_HIGH)
    grp_b = grp.astype(jnp.bfloat16)
    count = lax.dot_general(grp_b, mask_b, (((1,), (0,)), ((), ())),
                            preferred_element_type=jnp.float32)
    pw = s1 * (1.0 / _C)

    thr = jnp.mean(s1, axis=1, keepdims=True)
    binary = (s1 > thr).astype(jnp.float32)
    m_arr = pw * binary

    lane = lax.broadcasted_iota(jnp.int32, (4, 1024), 1)
    mx = jnp.max(m_arr, axis=1, keepdims=True)
    idx_max = jnp.min(jnp.where(m_arr == mx, lane, 1024), axis=1,
                      keepdims=True)

    ai = (idx_max // _H).astype(jnp.float32)
    aj = (idx_max % _H).astype(jnp.float32)
    pi = (lane // _H).astype(jnp.float32)
    pj = (lane % _H).astype(jnp.float32)
    ri = (pi - ai) * (1.0 / _H)
    rj = (pj - aj) * (1.0 / _H)
    dist = jnp.sqrt(ri * ri + rj * rj)
    ang = (jnp.arctan2(rj, ri) * (1.0 / math.pi) + 1.0) * 0.5

    cw = jnp.sum(pw * dist, axis=1, keepdims=True)
    ca = jnp.sum(pw * ang, axis=1, keepdims=True)
    pw2 = pw * pw
    p_pos = jnp.sum(jnp.where(pw > 0, pw2, 0.0), axis=1, keepdims=True)
    p_neg = jnp.sum(jnp.where(pw < 0, pw2, 0.0), axis=1, keepdims=True)

    w1 = w1_ref[...]
    t = cw * w1[0:1, :] + ca * w1[1:2, :]
    v = t * jnp.where(t > 0, p_pos, p_neg)
    w = lax.dot_general(v, w2_ref[...], (((1,), (0,)), ((), ())),
                        precision=_HIGH)
    pw_anchor = jnp.sum(jnp.where(lane == idx_max, pw, 0.0), axis=1,
                        keepdims=True)
    z = pw_anchor * w
    sinfo = jnp.where(z >= 0, z, 0.2 * z)
    row0_out[...] = row0_ref[...] + sinfo

    d_i = jnp.abs((p_row >> 5) - (p_col >> 5))
    d_j = jnp.abs((p_row & 31) - (p_col & 31))
    sten_b = jnp.where((d_i <= 1) & (d_j <= 1), (2 - d_i) * (2 - d_j),
                       0).astype(jnp.bfloat16)
    count_b = count.astype(jnp.bfloat16)
    csm = lax.dot_general(count_b, sten_b, (((1,), (0,)), ((), ())),
                          preferred_element_type=jnp.float32)
    ci = csm.astype(jnp.int32)
    key2 = ci * 1024 + (1023 - lane)

    keep = jnp.minimum(jnp.int32(_SELECT_NUM), sn_ref[0, 0])
    r_lane = lax.broadcasted_iota(jnp.int32, (1, _PAD_SEL), 1)
    pv = (lane[0:1, :] + 1).astype(jnp.float32)
    ones_bt = jnp.full((1, 1024), jnp.bfloat16(1))

    for b in range(_B):
        k2 = key2[b:b + 1, :]
        gtm = _bf01(k2 > k2.reshape(1024, 1))
        rank = lax.dot_general(gtm, ones_v, (((1,), (0,)), ((), ())),
                               preferred_element_type=jnp.float32)
        rank = rank.astype(jnp.int32)
        onehot = (rank == r_lane).astype(jnp.float32)
        patch = lax.dot_general(pv, onehot, (((1,), (0,)), ((), ())),
                                precision=_HIGH)
        patch = jnp.where(r_lane < keep, patch.astype(jnp.int32), 0)
        patch_out[b:b + 1, :] = patch
        gidx_out[b:b + 1, :] = patch + b * (_S + 1)


def _gather_body(tab_ref, idx_ref, out_ref, idx_v, rows_v, sem):
    nc = 2
    wid = lax.axis_index("s") * nc + lax.axis_index("c")
    per = (_B * _PAD_SEL) // (nc * 16)
    base = wid * per
    pltpu.sync_copy(idx_ref.at[pl.ds(base, per)], idx_v)
    pltpu.async_copy(tab_ref.at[idx_v], rows_v, sem).wait()
    pltpu.sync_copy(rows_v, out_ref.at[pl.ds(base, per)])


def kernel(hidden_states, x, contribution, select_num, W1, W2):
    del contribution
    score = x[:, :, 0, 1:].reshape(_B * _C, _S)
    row0 = hidden_states[:, 0, :]
    sn = jnp.asarray(select_num, jnp.int32).reshape(1, 1)
    hid_flat = hidden_states.reshape(_B * (_S + 1), _HIDDEN)

    row0_new, patch_pad, gidx = pl.pallas_call(
        _compute_body,
        in_specs=[pl.BlockSpec(memory_space=pltpu.SMEM)] +
                 [pl.BlockSpec(memory_space=pltpu.VMEM)] * 4,
        out_specs=[pl.BlockSpec(memory_space=pltpu.VMEM)] * 3,
        out_shape=[
            jax.ShapeDtypeStruct((_B, _HIDDEN), jnp.float32),
            jax.ShapeDtypeStruct((_B, _PAD_SEL), jnp.int32),
            jax.ShapeDtypeStruct((_B, _PAD_SEL), jnp.int32),
        ],
    )(sn, score, row0, W1, W2)

    hidden_out = hidden_states.at[:, 0, :].set(row0_new)

    mesh = plsc.VectorSubcoreMesh(core_axis_name="c", subcore_axis_name="s")
    gather = functools.partial(
        pl.kernel,
        mesh=mesh,
        out_type=jax.ShapeDtypeStruct((_B * _PAD_SEL, _HIDDEN), jnp.float32),
        scratch_types=[
            pltpu.VMEM(((_B * _PAD_SEL) // 32,), jnp.int32),
            pltpu.VMEM(((_B * _PAD_SEL) // 32, _HIDDEN), jnp.float32),
            pltpu.SemaphoreType.DMA,
        ],
    )(_gather_body)
    flat = gather(hid_flat, gidx.reshape(_B * _PAD_SEL))
    sel = flat.reshape(_B, _PAD_SEL, _HIDDEN)[:, :_SELECT_NUM, :]

    patch_idx = patch_pad[:, :_SELECT_NUM]
    selected = jnp.where((patch_idx == 0)[:, :, None],
                         hidden_out[:, 0, :][:, None, :], sel)
    return hidden_out, selected, patch_idx


# speedup vs baseline: 6.4828x; 1.0042x over previous
import functools
import math

import jax
import jax.numpy as jnp
from jax import lax
from jax.experimental import pallas as pl
from jax.experimental.pallas import tpu as pltpu
from jax.experimental.pallas import tpu_sc as plsc

_HIDDEN = 768
_PATCH_NUM = 84
_SELECT_NUM = 42
_B = 4
_C = 12
_S = 1024
_H = 32
_PAD_SEL = 64

_HIGH = lax.Precision.HIGHEST


def _bf01(pred):
    return pred.astype(jnp.float32).astype(jnp.bfloat16)


def _compute_body(sn_ref, score_ref, row0_ref, w1_ref, w2_ref,
                  row0_out, patch_out, gidx_out):
    score = score_ref[...]
    ones_v = jnp.full((1024, 1), jnp.bfloat16(1))

    u = lax.bitcast_convert_type(score, jnp.uint32)
    neg = (u >> jnp.uint32(31)) > jnp.uint32(0)
    ukey = jnp.where(neg, ~u, u | jnp.uint32(0x80000000))

    m = jnp.zeros((48, 1), jnp.uint32)
    for i in range(32):
        cand = m | jnp.uint32(0x80000000 >> i)
        cnt = jnp.sum((ukey >= cand).astype(jnp.int32), axis=1, keepdims=True)
        m = jnp.where(cnt >= _PATCH_NUM, cand, m)
    thr_key = m

    gt = ukey > thr_key
    eq = ukey == thr_key
    gtb = _bf01(gt)
    cnt_gt = lax.dot_general(gtb, ones_v, (((1,), (0,)), ((), ())),
                             preferred_element_type=jnp.float32)
    need = (float(_PATCH_NUM) - cnt_gt)

    p_row = lax.broadcasted_iota(jnp.int32, (1024, 1024), 0)
    p_col = lax.broadcasted_iota(jnp.int32, (1024, 1024), 1)
    slt = _bf01(p_row < p_col)

    eq_b = _bf01(eq)
    eq_rank = lax.dot_general(eq_b, slt, (((1,), (0,)), ((), ())),
                              preferred_element_type=jnp.float32)
    mask = gt | (eq & (eq_rank < need))
    mask_b = _bf01(mask)
    new_score = jnp.where(mask, score, score * 0.7)

    g_r = lax.broadcasted_iota(jnp.int32, (4, 48), 0)
    g_c = lax.broadcasted_iota(jnp.int32, (4, 48), 1)
    grp = jnp.where(g_c // _C == g_r, 1.0, 0.0).astype(jnp.float32)
    s1 = lax.dot_general(grp, new_score, (((1,), (0,)), ((), ())),
                         precision=_HIGH)
    grp_b = grp.astype(jnp.bfloat16)
    count = lax.dot_general(grp_b, mask_b, (((1,), (0,)), ((), ())),
                            preferred_element_type=jnp.float32)
    pw = s1 * (1.0 / _C)

    thr = jnp.mean(s1, axis=1, keepdims=True)
    binary = (s1 > thr).astype(jnp.float32)
    m_arr = pw * binary

    lane = lax.broadcasted_iota(jnp.int32, (4, 1024), 1)
    mx = jnp.max(m_arr, axis=1, keepdims=True)
    idx_max = jnp.min(jnp.where(m_arr == mx, lane, 1024), axis=1,
                      keepdims=True)

    ai = (idx_max // _H).astype(jnp.float32)
    aj = (idx_max % _H).astype(jnp.float32)
    pi = (lane // _H).astype(jnp.float32)
    pj = (lane % _H).astype(jnp.float32)
    ri = (pi - ai) * (1.0 / _H)
    rj = (pj - aj) * (1.0 / _H)
    dist = jnp.sqrt(ri * ri + rj * rj)
    ang = (jnp.arctan2(rj, ri) * (1.0 / math.pi) + 1.0) * 0.5

    cw = jnp.sum(pw * dist, axis=1, keepdims=True)
    ca = jnp.sum(pw * ang, axis=1, keepdims=True)
    pw2 = pw * pw
    p_pos = jnp.sum(jnp.where(pw > 0, pw2, 0.0), axis=1, keepdims=True)
    p_neg = jnp.sum(jnp.where(pw < 0, pw2, 0.0), axis=1, keepdims=True)

    w1 = w1_ref[...]
    t = cw * w1[0:1, :] + ca * w1[1:2, :]
    v = t * jnp.where(t > 0, p_pos, p_neg)
    w = lax.dot_general(v, w2_ref[...], (((1,), (0,)), ((), ())),
                        precision=_HIGH)
    pw_anchor = jnp.sum(jnp.where(lane == idx_max, pw, 0.0), axis=1,
                        keepdims=True)
    z = pw_anchor * w
    sinfo = jnp.where(z >= 0, z, 0.2 * z)
    row0_out[...] = row0_ref[...] + sinfo

    d_i = jnp.abs((p_row >> 5) - (p_col >> 5))
    d_j = jnp.abs((p_row & 31) - (p_col & 31))
    sten_b = jnp.where((d_i <= 1) & (d_j <= 1), (2 - d_i) * (2 - d_j),
                       0).astype(jnp.bfloat16)
    count_b = count.astype(jnp.bfloat16)
    csm = lax.dot_general(count_b, sten_b, (((1,), (0,)), ((), ())),
                          preferred_element_type=jnp.float32)
    ci = csm.astype(jnp.int32)
    key2 = ci * 1024 + (1023 - lane)

    keep = jnp.minimum(jnp.int32(_SELECT_NUM), sn_ref[0, 0])
    r_lane = lax.broadcasted_iota(jnp.int32, (1, _PAD_SEL), 1)
    pv = (lane[0:1, :] + 1).astype(jnp.float32)
    ones_bt = jnp.full((1, 1024), jnp.bfloat16(1))

    for b in range(_B):
        k2 = key2[b:b + 1, :]
        gtm = _bf01(k2 > k2.reshape(1024, 1))
        rank = lax.dot_general(gtm, ones_v, (((1,), (0,)), ((), ())),
                               preferred_element_type=jnp.float32)
        rank = rank.astype(jnp.int32)
        onehot = (rank == r_lane).astype(jnp.float32)
        patch = lax.dot_general(pv, onehot, (((1,), (0,)), ((), ())),
                                precision=_HIGH)
        patch = jnp.where(r_lane < keep, patch.astype(jnp.int32), 0)
        patch_out[b:b + 1, :] = patch
        gidx_out[b:b + 1, :] = patch + b * (_S + 1)


def _gather_body(tab_ref, idx_ref, out_ref, idx_v, rows_v, sem):
    nc = 2
    wid = lax.axis_index("s") * nc + lax.axis_index("c")
    per = (_B * _PAD_SEL) // (nc * 16)
    base = wid * per
    pltpu.sync_copy(idx_ref.at[pl.ds(base, per)], idx_v)
    pltpu.async_copy(tab_ref.at[idx_v], rows_v, sem).wait()
    pltpu.sync_copy(rows_v, out_ref.at[pl.ds(base, per)])


def kernel(hidden_states, x, contribution, select_num, W1, W2):
    del contribution
    score = x[:, :, 0, 1:].reshape(_B * _C, _S)
    row0 = hidden_states[:, 0, :]
    sn = jnp.asarray(select_num, jnp.int32).reshape(1, 1)
    hid_flat = hidden_states.reshape(_B * (_S + 1), _HIDDEN)

    row0_new, patch_pad, gidx = pl.pallas_call(
        _compute_body,
        in_specs=[pl.BlockSpec(memory_space=pltpu.SMEM)] +
                 [pl.BlockSpec(memory_space=pltpu.VMEM)] * 4,
        out_specs=[pl.BlockSpec(memory_space=pltpu.VMEM)] * 3,
        out_shape=[
            jax.ShapeDtypeStruct((_B, _HIDDEN), jnp.float32),
            jax.ShapeDtypeStruct((_B, _PAD_SEL), jnp.int32),
            jax.ShapeDtypeStruct((_B, _PAD_SEL), jnp.int32),
        ],
    )(sn, score, row0, W1, W2)

    hidden_out = hidden_states.at[:, 0, :].set(row0_new)

    mesh = plsc.VectorSubcoreMesh(core_axis_name="c", subcore_axis_name="s")
    gather = functools.partial(
        pl.kernel,
        mesh=mesh,
        out_type=jax.ShapeDtypeStruct((_B * _PAD_SEL, _HIDDEN), jnp.float32),
        scratch_types=[
            pltpu.VMEM(((_B * _PAD_SEL) // 32,), jnp.int32),
            pltpu.VMEM(((_B * _PAD_SEL) // 32, _HIDDEN), jnp.float32),
            pltpu.SemaphoreType.DMA,
        ],
    )(_gather_body)
    flat = gather(hid_flat, gidx.reshape(_B * _PAD_SEL))
    sel = flat.reshape(_B, _PAD_SEL, _HIDDEN)[:, :_SELECT_NUM, :]

    patch_idx = patch_pad[:, :_SELECT_NUM]
    selected = jnp.where((patch_idx == 0)[:, :, None],
                         hidden_out[:, 0, :][:, None, :], sel)
    return hidden_out, selected, patch_idx


# speedup vs baseline: 9.0330x; 1.3934x over previous
import functools
import math

import jax
import jax.numpy as jnp
from jax import lax
from jax.experimental import pallas as pl
from jax.experimental.pallas import tpu as pltpu
from jax.experimental.pallas import tpu_sc as plsc

_HIDDEN = 768
_PATCH_NUM = 84
_SELECT_NUM = 42
_B = 4
_C = 12
_S = 1024
_H = 32
_PAD_SEL = 64

_HIGH = lax.Precision.HIGHEST


def _bf01(pred):
    return pred.astype(jnp.float32).astype(jnp.bfloat16)


def _compute_body(sn_ref, score_ref, row0_ref, w1_ref, w2_ref,
                  row0_out, patch_out, gidx_out):
    score = score_ref[...]
    ones_v = jnp.full((1024, 1), jnp.bfloat16(1))

    u = lax.bitcast_convert_type(score, jnp.uint32)
    neg = (u >> jnp.uint32(31)) > jnp.uint32(0)
    ukey = jnp.where(neg, ~u, u | jnp.uint32(0x80000000))

    m = jnp.zeros((48, 1), jnp.uint32)
    for i in range(32):
        cand = m | jnp.uint32(0x80000000 >> i)
        cnt = jnp.sum((ukey >= cand).astype(jnp.int32), axis=1, keepdims=True)
        m = jnp.where(cnt >= _PATCH_NUM, cand, m)
    thr_key = m

    gt = ukey > thr_key
    eq = ukey == thr_key
    gtb = _bf01(gt)
    cnt_gt = lax.dot_general(gtb, ones_v, (((1,), (0,)), ((), ())),
                             preferred_element_type=jnp.float32)
    need = (float(_PATCH_NUM) - cnt_gt)

    p_row = lax.broadcasted_iota(jnp.int32, (1024, 1024), 0)
    p_col = lax.broadcasted_iota(jnp.int32, (1024, 1024), 1)
    slt = _bf01(p_row < p_col)

    eq_b = _bf01(eq)
    eq_rank = lax.dot_general(eq_b, slt, (((1,), (0,)), ((), ())),
                              preferred_element_type=jnp.float32)
    mask = gt | (eq & (eq_rank < need))
    mask_b = _bf01(mask)
    new_score = jnp.where(mask, score, score * 0.7)

    g_r = lax.broadcasted_iota(jnp.int32, (4, 48), 0)
    g_c = lax.broadcasted_iota(jnp.int32, (4, 48), 1)
    grp = jnp.where(g_c // _C == g_r, 1.0, 0.0).astype(jnp.float32)
    s1 = lax.dot_general(grp, new_score, (((1,), (0,)), ((), ())),
                         precision=_HIGH)
    grp_b = grp.astype(jnp.bfloat16)
    count = lax.dot_general(grp_b, mask_b, (((1,), (0,)), ((), ())),
                            preferred_element_type=jnp.float32)
    pw = s1 * (1.0 / _C)

    thr = jnp.mean(s1, axis=1, keepdims=True)
    binary = (s1 > thr).astype(jnp.float32)
    m_arr = pw * binary

    lane = lax.broadcasted_iota(jnp.int32, (4, 1024), 1)
    mx = jnp.max(m_arr, axis=1, keepdims=True)
    idx_max = jnp.min(jnp.where(m_arr == mx, lane, 1024), axis=1,
                      keepdims=True)

    ai = (idx_max // _H).astype(jnp.float32)
    aj = (idx_max % _H).astype(jnp.float32)
    pi = (lane // _H).astype(jnp.float32)
    pj = (lane % _H).astype(jnp.float32)
    ri = (pi - ai) * (1.0 / _H)
    rj = (pj - aj) * (1.0 / _H)
    dist = jnp.sqrt(ri * ri + rj * rj)
    ang = (jnp.arctan2(rj, ri) * (1.0 / math.pi) + 1.0) * 0.5

    cw = jnp.sum(pw * dist, axis=1, keepdims=True)
    ca = jnp.sum(pw * ang, axis=1, keepdims=True)
    pw2 = pw * pw
    p_pos = jnp.sum(jnp.where(pw > 0, pw2, 0.0), axis=1, keepdims=True)
    p_neg = jnp.sum(jnp.where(pw < 0, pw2, 0.0), axis=1, keepdims=True)

    w1 = w1_ref[...]
    t = cw * w1[0:1, :] + ca * w1[1:2, :]
    v = t * jnp.where(t > 0, p_pos, p_neg)
    w = lax.dot_general(v, w2_ref[...], (((1,), (0,)), ((), ())),
                        precision=_HIGH)
    pw_anchor = jnp.sum(jnp.where(lane == idx_max, pw, 0.0), axis=1,
                        keepdims=True)
    z = pw_anchor * w
    sinfo = jnp.where(z >= 0, z, 0.2 * z)
    row0_out[...] = row0_ref[...] + sinfo

    d_i = jnp.abs((p_row >> 5) - (p_col >> 5))
    d_j = jnp.abs((p_row & 31) - (p_col & 31))
    sten_b = jnp.where((d_i <= 1) & (d_j <= 1), (2 - d_i) * (2 - d_j),
                       0).astype(jnp.bfloat16)
    count_b = count.astype(jnp.bfloat16)
    csm = lax.dot_general(count_b, sten_b, (((1,), (0,)), ((), ())),
                          preferred_element_type=jnp.float32)
    ci = csm.astype(jnp.int32)
    key2 = ci * 1024 + (1023 - lane)

    keep = jnp.minimum(jnp.int32(_SELECT_NUM), sn_ref[0, 0])
    r_lane = lax.broadcasted_iota(jnp.int32, (1, _PAD_SEL), 1)
    pv = (lane[0:1, :] + 1).astype(jnp.float32)
    ones_bt = jnp.full((1, 1024), jnp.bfloat16(1))

    for b in range(_B):
        k2 = key2[b:b + 1, :]
        gtm = _bf01(k2 > k2.reshape(1024, 1))
        rank = lax.dot_general(gtm, ones_v, (((1,), (0,)), ((), ())),
                               preferred_element_type=jnp.float32)
        rank = rank.astype(jnp.int32)
        onehot = (rank == r_lane).astype(jnp.float32)
        patch = lax.dot_general(pv, onehot, (((1,), (0,)), ((), ())),
                                precision=_HIGH)
        patch = jnp.where(r_lane < keep, patch.astype(jnp.int32), 0)
        patch_out[b:b + 1, :] = patch
        gidx_out[b:b + 1, :] = patch + b * (_S + 1)


def _tc_gather_body(idx_ref, row0_ref, hid_ref, sel_ref):
    b = pl.program_id(0)
    rn = row0_ref[pl.ds(b, 1), :]
    for k in range(_PAD_SEL):
        idx = idx_ref[b, k]
        row = hid_ref[0, pl.ds(idx, 1), :]
        sel_ref[0, k:k + 1, :] = jnp.where(idx == 0, rn, row)


def _gather_body(tab_ref, idx_ref, out_ref, idx_v, rows_v, sem):
    nc = 2
    wid = lax.axis_index("s") * nc + lax.axis_index("c")
    per = (_B * _PAD_SEL) // (nc * 16)
    base = wid * per
    pltpu.sync_copy(idx_ref.at[pl.ds(base, per)], idx_v)
    pltpu.async_copy(tab_ref.at[idx_v], rows_v, sem).wait()
    pltpu.sync_copy(rows_v, out_ref.at[pl.ds(base, per)])


def kernel(hidden_states, x, contribution, select_num, W1, W2):
    del contribution
    score = x[:, :, 0, 1:].reshape(_B * _C, _S)
    row0 = hidden_states[:, 0, :]
    sn = jnp.asarray(select_num, jnp.int32).reshape(1, 1)
    hid_flat = hidden_states.reshape(_B * (_S + 1), _HIDDEN)

    row0_new, patch_pad, gidx = pl.pallas_call(
        _compute_body,
        in_specs=[pl.BlockSpec(memory_space=pltpu.SMEM)] +
                 [pl.BlockSpec(memory_space=pltpu.VMEM)] * 4,
        out_specs=[pl.BlockSpec(memory_space=pltpu.VMEM)] * 3,
        out_shape=[
            jax.ShapeDtypeStruct((_B, _HIDDEN), jnp.float32),
            jax.ShapeDtypeStruct((_B, _PAD_SEL), jnp.int32),
            jax.ShapeDtypeStruct((_B, _PAD_SEL), jnp.int32),
        ],
    )(sn, score, row0, W1, W2)

    hidden_out = hidden_states.at[:, 0, :].set(row0_new)

    sel = pl.pallas_call(
        _tc_gather_body,
        grid=(_B,),
        in_specs=[
            pl.BlockSpec(memory_space=pltpu.SMEM),
            pl.BlockSpec(memory_space=pltpu.VMEM),
            pl.BlockSpec((1, _S + 1, _HIDDEN), lambda b: (b, 0, 0)),
        ],
        out_specs=pl.BlockSpec((1, _PAD_SEL, _HIDDEN), lambda b: (b, 0, 0)),
        out_shape=jax.ShapeDtypeStruct((_B, _PAD_SEL, _HIDDEN), jnp.float32),
    )(patch_pad, row0_new, hidden_states)
    selected = sel[:, :_SELECT_NUM, :]
    patch_idx = patch_pad[:, :_SELECT_NUM]
    return hidden_out, selected, patch_idx
